# Initial kernel scaffold; baseline (speedup 1.0000x reference)
import jax
import jax.numpy as jnp
from jax.experimental import pallas as pl


def kernel(x0, t, noise, sqrt_alpha_bar, sqrt_one_minus_alpha_bar):
    raise NotImplementedError("write your pallas kernel here")


# speedup vs baseline: 3.7671x; 3.7671x over previous
import functools

import jax
import jax.numpy as jnp
from jax import lax
from jax.experimental import pallas as pl
from jax.experimental.pallas import tpu as pltpu
from jax.experimental.pallas import tpu_sc as plsc

N_ROWS = 16384
D = 128
T_LEN = 1000
N_WORKERS = 32
ROWS_PER_WORKER = N_ROWS // N_WORKERS

BR = 2048


def _sc_gather(table_a, table_b, t, out_a, out_b,
               ta_v, tb_v, idx_v, a_v, b_v):
    wid = lax.axis_index("s") * 2 + lax.axis_index("c")
    base = wid * ROWS_PER_WORKER
    pltpu.sync_copy(table_a, ta_v)
    pltpu.sync_copy(table_b, tb_v)
    pltpu.sync_copy(t.at[pl.ds(base, ROWS_PER_WORKER)], idx_v)
    for j in range(ROWS_PER_WORKER // 16):
        iv = idx_v[pl.ds(j * 16, 16)]
        a_v[pl.ds(j * 16, 16)] = plsc.load_gather(ta_v, [iv])
        b_v[pl.ds(j * 16, 16)] = plsc.load_gather(tb_v, [iv])
    pltpu.sync_copy(a_v, out_a.at[pl.ds(base, ROWS_PER_WORKER)])
    pltpu.sync_copy(b_v, out_b.at[pl.ds(base, ROWS_PER_WORKER)])


def _tc_fma(x_ref, n_ref, a_ref, b_ref, o_ref):
    o_ref[...] = a_ref[...] * x_ref[...] + b_ref[...] * n_ref[...]


@jax.jit
def kernel(x0, t, noise, sqrt_alpha_bar, sqrt_one_minus_alpha_bar):
    t32 = t.astype(jnp.int32)

    mesh = plsc.VectorSubcoreMesh(core_axis_name="c", subcore_axis_name="s")
    coef_a, coef_b = pl.kernel(
        _sc_gather,
        out_type=(
            jax.ShapeDtypeStruct((N_ROWS,), jnp.float32),
            jax.ShapeDtypeStruct((N_ROWS,), jnp.float32),
        ),
        mesh=mesh,
        compiler_params=pltpu.CompilerParams(needs_layout_passes=False),
        scratch_types=[
            pltpu.VMEM((T_LEN,), jnp.float32),
            pltpu.VMEM((T_LEN,), jnp.float32),
            pltpu.VMEM((ROWS_PER_WORKER,), jnp.int32),
            pltpu.VMEM((ROWS_PER_WORKER,), jnp.float32),
            pltpu.VMEM((ROWS_PER_WORKER,), jnp.float32),
        ],
    )(sqrt_alpha_bar, sqrt_one_minus_alpha_bar, t32)
    coef_a = coef_a.reshape(N_ROWS, 1)
    coef_b = coef_b.reshape(N_ROWS, 1)

    xt = pl.pallas_call(
        _tc_fma,
        grid=(N_ROWS // BR,),
        in_specs=[
            pl.BlockSpec((BR, D), lambda i: (i, 0)),
            pl.BlockSpec((BR, D), lambda i: (i, 0)),
            pl.BlockSpec((BR, 1), lambda i: (i, 0)),
            pl.BlockSpec((BR, 1), lambda i: (i, 0)),
        ],
        out_specs=pl.BlockSpec((BR, D), lambda i: (i, 0)),
        out_shape=jax.ShapeDtypeStruct((N_ROWS, D), jnp.float32),
    )(x0, noise, coef_a, coef_b)

    return (xt, noise)


# speedup vs baseline: 4.0421x; 1.0730x over previous
import functools

import jax
import jax.numpy as jnp
from jax import lax
from jax.experimental import pallas as pl
from jax.experimental.pallas import tpu as pltpu
from jax.experimental.pallas import tpu_sc as plsc

N_ROWS = 16384
D = 128
T_LEN = 1000
N_WORKERS = 32
ROWS_PER_WORKER = N_ROWS // N_WORKERS

BR = 2048


def _sc_gather(table_a, table_b, t, out_a, out_b,
               ta_v, tb_v, idx_v, a_v, b_v):
    wid = lax.axis_index("s") * 2 + lax.axis_index("c")
    base = wid * ROWS_PER_WORKER
    pltpu.sync_copy(table_a, ta_v)
    pltpu.sync_copy(table_b, tb_v)
    pltpu.sync_copy(t.at[pl.ds(base, ROWS_PER_WORKER)], idx_v)
    for j in range(ROWS_PER_WORKER // 16):
        iv = idx_v[pl.ds(j * 16, 16)]
        a_v[pl.ds(j * 16, 16)] = plsc.load_gather(ta_v, [iv])
        b_v[pl.ds(j * 16, 16)] = plsc.load_gather(tb_v, [iv])
    pltpu.sync_copy(a_v, out_a.at[pl.ds(base, ROWS_PER_WORKER)])
    pltpu.sync_copy(b_v, out_b.at[pl.ds(base, ROWS_PER_WORKER)])


def _tc_fma(x_ref, n_ref, a_ref, b_ref, o_ref, n_out_ref):
    n = n_ref[...]
    o_ref[...] = a_ref[...] * x_ref[...] + b_ref[...] * n
    n_out_ref[...] = n


@jax.jit
def kernel(x0, t, noise, sqrt_alpha_bar, sqrt_one_minus_alpha_bar):
    t32 = t.astype(jnp.int32)

    mesh = plsc.VectorSubcoreMesh(core_axis_name="c", subcore_axis_name="s")
    coef_a, coef_b = pl.kernel(
        _sc_gather,
        out_type=(
            jax.ShapeDtypeStruct((N_ROWS,), jnp.float32),
            jax.ShapeDtypeStruct((N_ROWS,), jnp.float32),
        ),
        mesh=mesh,
        compiler_params=pltpu.CompilerParams(needs_layout_passes=False),
        scratch_types=[
            pltpu.VMEM((T_LEN,), jnp.float32),
            pltpu.VMEM((T_LEN,), jnp.float32),
            pltpu.VMEM((ROWS_PER_WORKER,), jnp.int32),
            pltpu.VMEM((ROWS_PER_WORKER,), jnp.float32),
            pltpu.VMEM((ROWS_PER_WORKER,), jnp.float32),
        ],
    )(sqrt_alpha_bar, sqrt_one_minus_alpha_bar, t32)
    coef_a = coef_a.reshape(N_ROWS, 1)
    coef_b = coef_b.reshape(N_ROWS, 1)

    xt, noise_out = pl.pallas_call(
        _tc_fma,
        grid=(N_ROWS // BR,),
        in_specs=[
            pl.BlockSpec((BR, D), lambda i: (i, 0)),
            pl.BlockSpec((BR, D), lambda i: (i, 0)),
            pl.BlockSpec((BR, 1), lambda i: (i, 0)),
            pl.BlockSpec((BR, 1), lambda i: (i, 0)),
        ],
        out_specs=[
            pl.BlockSpec((BR, D), lambda i: (i, 0)),
            pl.BlockSpec((BR, D), lambda i: (i, 0)),
        ],
        out_shape=[
            jax.ShapeDtypeStruct((N_ROWS, D), jnp.float32),
            jax.ShapeDtypeStruct((N_ROWS, D), jnp.float32),
        ],
    )(x0, noise, coef_a, coef_b)

    return (xt, noise_out)
